# baseline (speedup 1.0000x reference)
import functools

import jax
import jax.numpy as jnp
from jax import lax
from jax.experimental import pallas as pl
from jax.experimental.pallas import tpu as pltpu
from jax.experimental.pallas import tpu_sc as plsc

B = 16384
F = 32
V = 1000
D = 128
R = 4
QC = 3 * F + D

NC = 2
NS = 16
NW = NC * NS
RW = B // NW
CE = 32
CW = CE * R
NCH = RW // CE

_mesh = plsc.VectorSubcoreMesh(core_axis_name="c", subcore_axis_name="s")


@functools.partial(
    pl.kernel,
    mesh=_mesh,
    out_type=jax.ShapeDtypeStruct((QC, B * R), jnp.float32),
    scratch_types=[
        pltpu.VMEM((RW,), jnp.int32),
        pltpu.VMEM((F, RW), jnp.float32),
        pltpu.VMEM((F, RW), jnp.float32),
        pltpu.VMEM((2, CE, D), jnp.float32),
        pltpu.VMEM((2, QC, CW), jnp.float32),
        pltpu.SemaphoreType.DMA,
        pltpu.SemaphoreType.DMA,
        pltpu.SemaphoreType.DMA,
        pltpu.SemaphoreType.DMA,
    ],
    compiler_params=pltpu.CompilerParams(needs_layout_passes=False),
)
def _gsp_query_sc(yt_hbm, xt_hbm, ids_hbm, table_hbm, out_hbm,
                  ids_v, y_v, x_v, emb_v, q_v,
                  gsem0, gsem1, ssem0, ssem1):
    gsem = (gsem0, gsem1)
    ssem = (ssem0, ssem1)
    wid = lax.axis_index("s") * NC + lax.axis_index("c")
    base = wid * RW
    cbase = base * R

    pltpu.sync_copy(ids_hbm.at[pl.ds(base, RW)], ids_v)
    pltpu.sync_copy(yt_hbm.at[:, pl.ds(base, RW)], y_v)
    pltpu.sync_copy(xt_hbm.at[:, pl.ds(base, RW)], x_v)

    ones16 = jnp.ones((16,), jnp.float32)
    for par in range(2):
        for c in range(F):
            for g in range(CW // 16):
                q_v[par, c, pl.ds(g * 16, 16)] = ones16

    iota = lax.iota(jnp.int32, 16)

    def gather_chunk(m, par):
        return pltpu.async_copy(
            table_hbm.at[ids_v.at[pl.ds(m * CE, CE)]],
            emb_v.at[par], gsem[par])

    gather_chunk(0, 0)

    def chunk_body(i, carry):
        for par in range(2):
            m = 2 * i + par
            if par == 0:
                gather_chunk(m + 1, 1)
            else:
                @pl.when(i < (NCH // 2) - 1)
                def _():
                    gather_chunk(m + 1, 0)
            pltpu.make_async_copy(table_hbm.at[pl.ds(0, CE)],
                                  emb_v.at[par], gsem[par]).wait()
            @pl.when(i > 0)
            def _():
                pltpu.make_async_copy(out_hbm.at[:, pl.ds(0, CW)],
                                      q_v.at[par], ssem[par]).wait()
            ex0 = m * CE
            for g in range(CW // 16):
                cidx = ex0 + ((g * 16 + iota) >> 2)
                for f in range(F):
                    fidx = jnp.full((16,), f, jnp.int32)
                    q_v[par, F + f, pl.ds(g * 16, 16)] = (
                        plsc.load_gather(y_v, [fidx, cidx]))
                    q_v[par, 2 * F + f, pl.ds(g * 16, 16)] = (
                        plsc.load_gather(x_v, [fidx, cidx]))
            pidx = jnp.full((16,), par, jnp.int32)

            def emb_row(d, c):
                didx = jnp.full((16,), 0, jnp.int32) + d
                for g in range(CW // 16):
                    eoff = (g * 16 + iota) >> 2
                    q_v[par, 3 * F + d, pl.ds(g * 16, 16)] = (
                        plsc.load_gather(emb_v, [pidx, eoff, didx]))
                return c

            lax.fori_loop(0, D, emb_row, 0)
            pltpu.async_copy(q_v.at[par],
                             out_hbm.at[:, pl.ds(cbase + m * CW, CW)],
                             ssem[par])
        return carry

    lax.fori_loop(0, NCH // 2, chunk_body, 0)

    for par in range(2):
        pltpu.make_async_copy(out_hbm.at[:, pl.ds(0, CW)],
                              q_v.at[par], ssem[par]).wait()


def kernel(gsp_y_osgb_fourier, gsp_x_osgb_fourier, hrvsatellite_solar_azimuth,
           gsp_id, emb_table):
    yt = jnp.transpose(gsp_y_osgb_fourier[:, 0, :])
    xt = jnp.transpose(gsp_x_osgb_fourier[:, 0, :])
    ids = gsp_id[:, 0]
    n_repeats = hrvsatellite_solar_azimuth.shape[0] // B
    assert n_repeats == R
    out_t = _gsp_query_sc(yt, xt, ids, emb_table)
    return jnp.transpose(out_t)[:, None, :]


# speedup vs baseline: 1.2178x; 1.2178x over previous
import functools

import jax
import jax.numpy as jnp
from jax import lax
from jax.experimental import pallas as pl
from jax.experimental.pallas import tpu as pltpu
from jax.experimental.pallas import tpu_sc as plsc

B = 16384
F = 32
V = 1000
D = 128
R = 4
QC = 3 * F + D

NC = 2
NS = 16
NW = NC * NS
RW = B // NW
CE = 32
CW = CE * R
NCH = RW // CE

_mesh = plsc.VectorSubcoreMesh(core_axis_name="c", subcore_axis_name="s")


@functools.partial(
    pl.kernel,
    mesh=_mesh,
    out_type=jax.ShapeDtypeStruct((QC, B * R), jnp.float32),
    scratch_types=[
        pltpu.VMEM((RW,), jnp.int32),
        pltpu.VMEM((F, RW), jnp.float32),
        pltpu.VMEM((F, RW), jnp.float32),
        pltpu.VMEM((2, CE, D), jnp.float32),
        pltpu.VMEM((2, QC, CW), jnp.float32),
        pltpu.SemaphoreType.DMA,
        pltpu.SemaphoreType.DMA,
        pltpu.SemaphoreType.DMA,
        pltpu.SemaphoreType.DMA,
    ],
    compiler_params=pltpu.CompilerParams(needs_layout_passes=False),
)
def _gsp_query_sc(yt_hbm, xt_hbm, ids_hbm, table_hbm, out_hbm,
                  ids_v, y_v, x_v, emb_v, q_v,
                  gsem0, gsem1, ssem0, ssem1):
    gsem = (gsem0, gsem1)
    ssem = (ssem0, ssem1)
    wid = lax.axis_index("s") * NC + lax.axis_index("c")
    base = wid * RW
    cbase = base * R

    pltpu.sync_copy(ids_hbm.at[pl.ds(base, RW)], ids_v)
    pltpu.sync_copy(yt_hbm.at[:, pl.ds(base, RW)], y_v)
    pltpu.sync_copy(xt_hbm.at[:, pl.ds(base, RW)], x_v)

    ones16 = jnp.ones((16,), jnp.float32)
    for par in range(2):
        for c in range(F):
            for g in range(CW // 16):
                q_v[par, c, pl.ds(g * 16, 16)] = ones16

    iota = lax.iota(jnp.int32, 16)

    def gather_chunk(m, par):
        return pltpu.async_copy(
            table_hbm.at[ids_v.at[pl.ds(m * CE, CE)]],
            emb_v.at[par], gsem[par])

    gather_chunk(0, 0)

    def chunk_body(i, carry):
        for par in range(2):
            m = 2 * i + par
            if par == 0:
                gather_chunk(m + 1, 1)
            else:
                @pl.when(i < (NCH // 2) - 1)
                def _():
                    gather_chunk(m + 1, 0)
            pltpu.make_async_copy(table_hbm.at[pl.ds(0, CE)],
                                  emb_v.at[par], gsem[par]).wait()
            @pl.when(i > 0)
            def _():
                pltpu.make_async_copy(out_hbm.at[:, pl.ds(0, CW)],
                                      q_v.at[par], ssem[par]).wait()
            ex0 = m * CE
            for g in range(CW // 16):
                cidx = ex0 + ((g * 16 + iota) >> 2)
                for f in range(F):
                    fidx = jnp.full((16,), f, jnp.int32)
                    q_v[par, F + f, pl.ds(g * 16, 16)] = (
                        plsc.load_gather(y_v, [fidx, cidx]))
                    q_v[par, 2 * F + f, pl.ds(g * 16, 16)] = (
                        plsc.load_gather(x_v, [fidx, cidx]))
            pidx = jnp.full((16,), par, jnp.int32)
            rows_c = [(3 * F + c) + iota for c in range(0, D, 16)]
            for e in range(CE):
                evs = [emb_v[par, e, pl.ds(c, 16)] for c in range(0, D, 16)]
                for r in range(R):
                    cols = jnp.full((16,), R * e + r, jnp.int32)
                    for ci in range(D // 16):
                        plsc.store_scatter(q_v, [pidx, rows_c[ci], cols],
                                           evs[ci])
            pltpu.async_copy(q_v.at[par],
                             out_hbm.at[:, pl.ds(cbase + m * CW, CW)],
                             ssem[par])
        return carry

    lax.fori_loop(0, NCH // 2, chunk_body, 0)

    for par in range(2):
        pltpu.make_async_copy(out_hbm.at[:, pl.ds(0, CW)],
                              q_v.at[par], ssem[par]).wait()


def kernel(gsp_y_osgb_fourier, gsp_x_osgb_fourier, hrvsatellite_solar_azimuth,
           gsp_id, emb_table):
    yt = jnp.transpose(gsp_y_osgb_fourier[:, 0, :])
    xt = jnp.transpose(gsp_x_osgb_fourier[:, 0, :])
    ids = gsp_id[:, 0]
    n_repeats = hrvsatellite_solar_azimuth.shape[0] // B
    assert n_repeats == R
    out_t = _gsp_query_sc(yt, xt, ids, emb_table)
    return jnp.transpose(out_t)[:, None, :]


# speedup vs baseline: 1.2491x; 1.0257x over previous
import functools

import jax
import jax.numpy as jnp
from jax import lax
from jax.experimental import pallas as pl
from jax.experimental.pallas import tpu as pltpu
from jax.experimental.pallas import tpu_sc as plsc

B = 16384
F = 32
V = 1000
D = 128
R = 4
QC = 3 * F + D

NC = 2
NS = 16
NW = NC * NS
RW = B // NW
CE = 32
CW = CE * R
NCH = RW // CE

_mesh = plsc.VectorSubcoreMesh(core_axis_name="c", subcore_axis_name="s")


@functools.partial(
    pl.kernel,
    mesh=_mesh,
    out_type=jax.ShapeDtypeStruct((QC, B * R), jnp.float32),
    scratch_types=[
        pltpu.VMEM((RW,), jnp.int32),
        pltpu.VMEM((F, RW), jnp.float32),
        pltpu.VMEM((F, RW), jnp.float32),
        pltpu.VMEM((2, CE, D), jnp.float32),
        pltpu.VMEM((2, QC, CW + 1), jnp.float32),
        pltpu.SemaphoreType.DMA,
        pltpu.SemaphoreType.DMA,
        pltpu.SemaphoreType.DMA,
        pltpu.SemaphoreType.DMA,
    ],
    compiler_params=pltpu.CompilerParams(needs_layout_passes=False,
                                         use_tc_tiling_on_sc=False),
)
def _gsp_query_sc(yt_hbm, xt_hbm, ids_hbm, table_hbm, out_hbm,
                  ids_v, y_v, x_v, emb_v, q_v,
                  gsem0, gsem1, ssem0, ssem1):
    gsem = (gsem0, gsem1)
    ssem = (ssem0, ssem1)
    wid = lax.axis_index("s") * NC + lax.axis_index("c")
    base = wid * RW
    cbase = base * R

    pltpu.sync_copy(ids_hbm.at[pl.ds(base, RW)], ids_v)
    pltpu.sync_copy(yt_hbm.at[:, pl.ds(base, RW)], y_v)
    pltpu.sync_copy(xt_hbm.at[:, pl.ds(base, RW)], x_v)

    ones16 = jnp.ones((16,), jnp.float32)
    for par in range(2):
        for c in range(F):
            for g in range(CW // 16):
                q_v[par, c, pl.ds(g * 16, 16)] = ones16

    iota = lax.iota(jnp.int32, 16)

    def gather_chunk(m, par):
        return pltpu.async_copy(
            table_hbm.at[ids_v.at[pl.ds(m * CE, CE)]],
            emb_v.at[par], gsem[par])

    gather_chunk(0, 0)

    def chunk_body(i, carry):
        for par in range(2):
            m = 2 * i + par
            if par == 0:
                gather_chunk(m + 1, 1)
            else:
                @pl.when(i < (NCH // 2) - 1)
                def _():
                    gather_chunk(m + 1, 0)
            pltpu.make_async_copy(table_hbm.at[pl.ds(0, CE)],
                                  emb_v.at[par], gsem[par]).wait()
            @pl.when(i > 0)
            def _():
                pltpu.make_async_copy(out_hbm.at[:, pl.ds(0, CW)],
                                      q_v.at[par, :, pl.ds(0, CW)],
                                      ssem[par]).wait()
            ex0 = m * CE
            for g in range(CW // 16):
                cidx = ex0 + ((g * 16 + iota) >> 2)
                for f in range(F):
                    fidx = jnp.full((16,), f, jnp.int32)
                    q_v[par, F + f, pl.ds(g * 16, 16)] = (
                        plsc.load_gather(y_v, [fidx, cidx]))
                    q_v[par, 2 * F + f, pl.ds(g * 16, 16)] = (
                        plsc.load_gather(x_v, [fidx, cidx]))
            pidx = jnp.full((16,), par, jnp.int32)
            for c in range(0, D, 16):
                rows = (3 * F + c) + iota
                for e in range(CE):
                    ev = emb_v[par, e, pl.ds(c, 16)]
                    for r in range(R):
                        cols = jnp.full((16,), R * e + r, jnp.int32)
                        plsc.store_scatter(q_v, [pidx, rows, cols], ev)
            pltpu.async_copy(q_v.at[par, :, pl.ds(0, CW)],
                             out_hbm.at[:, pl.ds(cbase + m * CW, CW)],
                             ssem[par])
        return carry

    lax.fori_loop(0, NCH // 2, chunk_body, 0)

    for par in range(2):
        pltpu.make_async_copy(out_hbm.at[:, pl.ds(0, CW)],
                              q_v.at[par, :, pl.ds(0, CW)],
                              ssem[par]).wait()


def kernel(gsp_y_osgb_fourier, gsp_x_osgb_fourier, hrvsatellite_solar_azimuth,
           gsp_id, emb_table):
    yt = jnp.transpose(gsp_y_osgb_fourier[:, 0, :])
    xt = jnp.transpose(gsp_x_osgb_fourier[:, 0, :])
    ids = gsp_id[:, 0]
    n_repeats = hrvsatellite_solar_azimuth.shape[0] // B
    assert n_repeats == R
    out_t = _gsp_query_sc(yt, xt, ids, emb_table)
    return jnp.transpose(out_t)[:, None, :]


# speedup vs baseline: 1.3734x; 1.0995x over previous
import functools

import jax
import jax.numpy as jnp
from jax import lax
from jax.experimental import pallas as pl
from jax.experimental.pallas import tpu as pltpu
from jax.experimental.pallas import tpu_sc as plsc

B = 16384
F = 32
V = 1000
D = 128
R = 4
QC = 3 * F + D

NC = 2
NS = 16
NW = NC * NS
RW = B // NW
CE = 32
NCH = RW // CE

_mesh = plsc.VectorSubcoreMesh(core_axis_name="c", subcore_axis_name="s")


@functools.partial(
    pl.kernel,
    mesh=_mesh,
    out_type=jax.ShapeDtypeStruct((B, R, QC), jnp.float32),
    scratch_types=[
        pltpu.VMEM((RW,), jnp.int32),
        pltpu.VMEM((RW, F), jnp.float32),
        pltpu.VMEM((RW, F), jnp.float32),
        pltpu.VMEM((2, CE, QC), jnp.float32),
        pltpu.VMEM((2, CE, D), jnp.float32),
        pltpu.SemaphoreType.DMA,
        pltpu.SemaphoreType.DMA,
        pltpu.SemaphoreType.DMA,
        pltpu.SemaphoreType.DMA,
    ],
    compiler_params=pltpu.CompilerParams(needs_layout_passes=False,
                                         use_tc_tiling_on_sc=False),
)
def _gsp_query_sc(ys_hbm, xs_hbm, ids_hbm, table_hbm, out_hbm,
                  ids_v, y_v, x_v, blk_v, emb_v,
                  gsem0, gsem1, ssem0, ssem1):
    gsem = (gsem0, gsem1)
    ssem = (ssem0, ssem1)
    wid = lax.axis_index("s") * NC + lax.axis_index("c")
    base = wid * RW

    pltpu.sync_copy(ids_hbm.at[pl.ds(base, RW)], ids_v)
    pltpu.sync_copy(ys_hbm.at[pl.ds(base, RW), :], y_v)
    pltpu.sync_copy(xs_hbm.at[pl.ds(base, RW), :], x_v)

    ones16 = jnp.ones((16,), jnp.float32)
    for par in range(2):
        for e in range(CE):
            blk_v[par, e, pl.ds(0, 16)] = ones16
            blk_v[par, e, pl.ds(16, 16)] = ones16

    def gather_chunk(m, par):
        return pltpu.async_copy(
            table_hbm.at[ids_v.at[pl.ds(m * CE, CE)]],
            emb_v.at[par], gsem[par])

    def wait_out(par):
        for r in range(R):
            pltpu.make_async_copy(
                blk_v.at[par],
                out_hbm.at[pl.ds(0, CE), r, :], ssem[par]).wait()

    gather_chunk(0, 0)

    def chunk_body(i, carry):
        for par in range(2):
            m = 2 * i + par
            @pl.when(i > 0)
            def _():
                wait_out(par)
            if par == 0:
                gather_chunk(m + 1, 1)
            else:
                @pl.when(i < (NCH // 2) - 1)
                def _():
                    gather_chunk(m + 1, 0)
            ex0 = m * CE
            for e in range(CE):
                blk_v[par, e, pl.ds(F, 16)] = y_v[ex0 + e, pl.ds(0, 16)]
                blk_v[par, e, pl.ds(F + 16, 16)] = y_v[ex0 + e, pl.ds(16, 16)]
                blk_v[par, e, pl.ds(2 * F, 16)] = x_v[ex0 + e, pl.ds(0, 16)]
                blk_v[par, e, pl.ds(2 * F + 16, 16)] = x_v[ex0 + e,
                                                           pl.ds(16, 16)]
            pltpu.make_async_copy(table_hbm.at[pl.ds(0, CE)],
                                  emb_v.at[par], gsem[par]).wait()
            for e in range(CE):
                for c in range(0, D, 16):
                    blk_v[par, e, pl.ds(3 * F + c, 16)] = (
                        emb_v[par, e, pl.ds(c, 16)])
            for r in range(R):
                pltpu.async_copy(
                    blk_v.at[par],
                    out_hbm.at[pl.ds(base + m * CE, CE), r, :], ssem[par])
        return carry

    lax.fori_loop(0, NCH // 2, chunk_body, 0)

    for par in range(2):
        wait_out(par)


def kernel(gsp_y_osgb_fourier, gsp_x_osgb_fourier, hrvsatellite_solar_azimuth,
           gsp_id, emb_table):
    ys = gsp_y_osgb_fourier[:, 0, :]
    xs = gsp_x_osgb_fourier[:, 0, :]
    ids = gsp_id[:, 0]
    n_repeats = hrvsatellite_solar_azimuth.shape[0] // B
    assert n_repeats == R
    out = _gsp_query_sc(ys, xs, ids, emb_table)
    return out.reshape(B * R, 1, QC)


# speedup vs baseline: 1.3734x; 1.0000x over previous
import functools

import jax
import jax.numpy as jnp
from jax import lax
from jax.experimental import pallas as pl
from jax.experimental.pallas import tpu as pltpu
from jax.experimental.pallas import tpu_sc as plsc

B = 16384
F = 32
V = 1000
D = 128
R = 4
QC = 3 * F + D

NC = 2
NS = 16
NW = NC * NS
RW = B // NW
CE = 64
NCH = RW // CE

_mesh = plsc.VectorSubcoreMesh(core_axis_name="c", subcore_axis_name="s")


@functools.partial(
    pl.kernel,
    mesh=_mesh,
    out_type=jax.ShapeDtypeStruct((B, R, QC), jnp.float32),
    scratch_types=[
        pltpu.VMEM((RW,), jnp.int32),
        pltpu.VMEM((RW, F), jnp.float32),
        pltpu.VMEM((RW, F), jnp.float32),
        pltpu.VMEM((2, CE, QC), jnp.float32),
        pltpu.VMEM((2, CE, D), jnp.float32),
        pltpu.SemaphoreType.DMA,
        pltpu.SemaphoreType.DMA,
        pltpu.SemaphoreType.DMA,
        pltpu.SemaphoreType.DMA,
    ],
    compiler_params=pltpu.CompilerParams(needs_layout_passes=False,
                                         use_tc_tiling_on_sc=False),
)
def _gsp_query_sc(ys_hbm, xs_hbm, ids_hbm, table_hbm, out_hbm,
                  ids_v, y_v, x_v, blk_v, emb_v,
                  gsem0, gsem1, ssem0, ssem1):
    gsem = (gsem0, gsem1)
    ssem = (ssem0, ssem1)
    wid = lax.axis_index("s") * NC + lax.axis_index("c")
    base = wid * RW

    pltpu.sync_copy(ids_hbm.at[pl.ds(base, RW)], ids_v)
    pltpu.sync_copy(ys_hbm.at[pl.ds(base, RW), :], y_v)
    pltpu.sync_copy(xs_hbm.at[pl.ds(base, RW), :], x_v)

    ones16 = jnp.ones((16,), jnp.float32)
    for par in range(2):
        for e in range(CE):
            blk_v[par, e, pl.ds(0, 16)] = ones16
            blk_v[par, e, pl.ds(16, 16)] = ones16

    def gather_chunk(m, par):
        return pltpu.async_copy(
            table_hbm.at[ids_v.at[pl.ds(m * CE, CE)]],
            emb_v.at[par], gsem[par])

    def wait_out(par):
        for r in range(R):
            pltpu.make_async_copy(
                blk_v.at[par],
                out_hbm.at[pl.ds(0, CE), r, :], ssem[par]).wait()

    gather_chunk(0, 0)

    def chunk_body(i, carry):
        for par in range(2):
            m = 2 * i + par
            @pl.when(i > 0)
            def _():
                wait_out(par)
            if par == 0:
                gather_chunk(m + 1, 1)
            else:
                @pl.when(i < (NCH // 2) - 1)
                def _():
                    gather_chunk(m + 1, 0)
            ex0 = m * CE
            for e in range(CE):
                blk_v[par, e, pl.ds(F, 16)] = y_v[ex0 + e, pl.ds(0, 16)]
                blk_v[par, e, pl.ds(F + 16, 16)] = y_v[ex0 + e, pl.ds(16, 16)]
                blk_v[par, e, pl.ds(2 * F, 16)] = x_v[ex0 + e, pl.ds(0, 16)]
                blk_v[par, e, pl.ds(2 * F + 16, 16)] = x_v[ex0 + e,
                                                           pl.ds(16, 16)]
            pltpu.make_async_copy(table_hbm.at[pl.ds(0, CE)],
                                  emb_v.at[par], gsem[par]).wait()
            for e in range(CE):
                for c in range(0, D, 16):
                    blk_v[par, e, pl.ds(3 * F + c, 16)] = (
                        emb_v[par, e, pl.ds(c, 16)])
            for r in range(R):
                pltpu.async_copy(
                    blk_v.at[par],
                    out_hbm.at[pl.ds(base + m * CE, CE), r, :], ssem[par])
        return carry

    lax.fori_loop(0, NCH // 2, chunk_body, 0)

    for par in range(2):
        wait_out(par)


def kernel(gsp_y_osgb_fourier, gsp_x_osgb_fourier, hrvsatellite_solar_azimuth,
           gsp_id, emb_table):
    ys = gsp_y_osgb_fourier[:, 0, :]
    xs = gsp_x_osgb_fourier[:, 0, :]
    ids = gsp_id[:, 0]
    n_repeats = hrvsatellite_solar_azimuth.shape[0] // B
    assert n_repeats == R
    out = _gsp_query_sc(ys, xs, ids, emb_table)
    return out.reshape(B * R, 1, QC)
